# baseline (speedup 1.0000x reference)
import functools

import jax
import jax.numpy as jnp
from jax import lax
from jax.experimental import pallas as pl
from jax.experimental.pallas import tpu as pltpu
from jax.experimental.pallas import tpu_sc as plsc

N = 10000
E = 320000
H = 128
NUM_RBF = 16
CUTOFF = 5.0
LOG2 = 0.6931471805599453

NC = 2
NS = 16
L = 16
NW = NC * NS
C = 32
E_PAD = 327680
EPT = E_PAD // NW
NCH = EPT // C
N_PAD = 10240
RPT = N_PAD // NS

BE = 2000
BN = 2000


def _wf_body(ea_ref, ew_ref, w1_ref, b1_ref, w2_ref, b2_ref, o_ref):
    ea = ea_ref[...].astype(jnp.bfloat16)
    v = jnp.dot(ea, w1_ref[...].astype(jnp.bfloat16),
                preferred_element_type=jnp.float32)
    v = v + b1_ref[...]
    v = jnp.maximum(v, 0.0) + jnp.log1p(jnp.exp(-jnp.abs(v))) - LOG2
    wf = jnp.dot(v.astype(jnp.bfloat16), w2_ref[...].astype(jnp.bfloat16),
                 preferred_element_type=jnp.float32)
    wf = wf + b2_ref[...]
    cut = 0.5 * (jnp.cos(ew_ref[...] * (jnp.pi / CUTOFF)) + 1.0)
    o_ref[...] = wf * cut


def _h_body(x_ref, w_ref, o_ref):
    o_ref[...] = jnp.dot(x_ref[...], w_ref[...],
                         preferred_element_type=jnp.float32)


def _out_body(p0_ref, p1_ref, w2_ref, b2_ref, wo_ref, bo_ref, o_ref):
    agg = p0_ref[...] + p1_ref[...]
    h2 = jnp.dot(agg, w2_ref[...], preferred_element_type=jnp.float32)
    h2 = jnp.tanh(h2 + b2_ref[...])
    o_ref[...] = jnp.dot(h2, wo_ref[...],
                         preferred_element_type=jnp.float32) + bo_ref[...]


_sc_mesh = plsc.VectorSubcoreMesh(core_axis_name="c", subcore_axis_name="s")


NBUF = 2


@functools.partial(
    pl.kernel,
    out_type=[jax.ShapeDtypeStruct((N_PAD, H), jnp.float32),
              jax.ShapeDtypeStruct((N_PAD, H), jnp.float32)],
    mesh=_sc_mesh,
    scratch_types=[
        pltpu.VMEM((EPT // 128, 128), jnp.int32),
        pltpu.VMEM((EPT // 128, 128), jnp.int32),
        pltpu.VMEM((NBUF, C, H), jnp.float32),
        pltpu.VMEM((NBUF, C, H), jnp.float32),
        pltpu.VMEM((NBUF, C, H), jnp.float32),
        pltpu.VMEM((NBUF, C), jnp.int32),
        pltpu.VMEM_SHARED((N_PAD, H), jnp.float32),
        pltpu.SemaphoreType.DMA((NBUF,)),
        pltpu.SemaphoreType.DMA((NBUF,)),
        pltpu.SemaphoreType.DMA((NBUF,)),
    ],
)
def _sc_scatter(h_hbm, wf_hbm, src_hbm, dst_hbm, out0_hbm, out1_hbm,
                src_v, dst_v, rows_v, wf_v, msg_v, dcur_v, agg_sh,
                gsem, wsem, ssem):
    c = lax.axis_index("c")
    s = lax.axis_index("s")
    w = c * NS + s

    idxr = EPT // 128
    pltpu.sync_copy(src_hbm.at[pl.ds(w * idxr, idxr)], src_v)
    pltpu.sync_copy(dst_hbm.at[pl.ds(w * idxr, idxr)], dst_v)

    cpr = 128 // C

    def idx_slice(v, i):
        return v.at[i // cpr, pl.ds((i % cpr) * C, C)]

    zbuf = rows_v.at[0]

    @pl.loop(0, C)
    def _zero_rows(r):
        for j in range(H // L):
            zbuf.at[pl.ds(r, 1), pl.ds(j * L, L)][...] = jnp.zeros(
                (1, L), jnp.float32)

    base_n = s * RPT
    for k in range(RPT // C):
        pltpu.sync_copy(zbuf, agg_sh.at[pl.ds(base_n + k * C, C)])

    def issue(i, b):
        pltpu.async_copy(h_hbm.at[idx_slice(src_v, i)], rows_v.at[b],
                         gsem.at[b])
        pltpu.async_copy(wf_hbm.at[pl.ds((w * NCH + i) * C, C)],
                         wf_v.at[b], wsem.at[b])

    def wait_in(b):
        pltpu.make_async_copy(h_hbm.at[idx_slice(src_v, 0)], rows_v.at[b],
                              gsem.at[b]).wait()
        pltpu.make_async_copy(wf_hbm.at[pl.ds(0, C)], wf_v.at[b],
                              wsem.at[b]).wait()

    def wait_scatter(b):
        pltpu.make_async_copy(msg_v.at[b], agg_sh.at[dcur_v.at[b]],
                              ssem.at[b]).wait()

    issue(0, 0)
    issue(1, 1)
    plsc.subcore_barrier()

    @pl.loop(0, NCH // NBUF)
    def _edge_step(t):
        i0 = t * NBUF
        for b in range(NBUF):
            i = i0 + b
            wait_in(b)

            @pl.when(t > 0)
            def _():
                wait_scatter(b)

            rb, wb, mb = rows_v.at[b], wf_v.at[b], msg_v.at[b]

            @pl.loop(0, C)
            def _mul_row(r):
                for j in range(H // L):
                    slc = (pl.ds(r, 1), pl.ds(j * L, L))
                    mb.at[*slc][...] = rb.at[*slc][...] * wb.at[*slc][...]

            @pl.when(i + NBUF < NCH)
            def _():
                issue(i + NBUF, b)

            dslc = idx_slice(dst_v, i)
            for j in range(C // L):
                dcur_v.at[b, pl.ds(j * L, L)][...] = \
                    dslc.at[pl.ds(j * L, L)][...]
            pltpu.async_copy(msg_v.at[b], agg_sh.at[dcur_v.at[b]],
                             ssem.at[b], add=True)

    for b in range(NBUF):
        wait_scatter(b)
    plsc.subcore_barrier()

    @pl.when(c == 0)
    def _():
        pltpu.sync_copy(agg_sh.at[pl.ds(base_n, RPT)],
                        out0_hbm.at[pl.ds(base_n, RPT)])

    @pl.when(c == 1)
    def _():
        pltpu.sync_copy(agg_sh.at[pl.ds(base_n, RPT)],
                        out1_hbm.at[pl.ds(base_n, RPT)])


def kernel(x, edge_index, edge_weight, edge_attr, atom_types, seq_neighs,
           lin1_w, fnet_w1, fnet_b1, fnet_w2, fnet_b2, lin2_w, lin2_b,
           lin_w, lin_b):
    pad = E_PAD - E
    src = jnp.pad(edge_index[0], (0, pad)).reshape(E_PAD // 128, 128)
    dst = jnp.pad(edge_index[1], (0, pad),
                  constant_values=N).reshape(E_PAD // 128, 128)
    ew = edge_weight.reshape(E, 1)

    h = pl.pallas_call(
        _h_body,
        out_shape=jax.ShapeDtypeStruct((N, H), jnp.float32),
    )(x, lin1_w)

    wf = pl.pallas_call(
        _wf_body,
        grid=(E // BE,),
        in_specs=[
            pl.BlockSpec((BE, NUM_RBF), lambda i: (i, 0)),
            pl.BlockSpec((BE, 1), lambda i: (i, 0)),
            pl.BlockSpec((NUM_RBF, H), lambda i: (0, 0)),
            pl.BlockSpec((1, H), lambda i: (0, 0)),
            pl.BlockSpec((H, H), lambda i: (0, 0)),
            pl.BlockSpec((1, H), lambda i: (0, 0)),
        ],
        out_specs=pl.BlockSpec((BE, H), lambda i: (i, 0)),
        out_shape=jax.ShapeDtypeStruct((E_PAD, H), jnp.float32),
    )(edge_attr, ew, fnet_w1, fnet_b1.reshape(1, H),
      fnet_w2, fnet_b2.reshape(1, H))

    p0, p1 = _sc_scatter(h, wf, src, dst)

    out = pl.pallas_call(
        _out_body,
        grid=(N // BN,),
        in_specs=[
            pl.BlockSpec((BN, H), lambda i: (i, 0)),
            pl.BlockSpec((BN, H), lambda i: (i, 0)),
            pl.BlockSpec((H, H), lambda i: (0, 0)),
            pl.BlockSpec((1, H), lambda i: (0, 0)),
            pl.BlockSpec((H, H), lambda i: (0, 0)),
            pl.BlockSpec((1, H), lambda i: (0, 0)),
        ],
        out_specs=pl.BlockSpec((BN, H), lambda i: (i, 0)),
        out_shape=jax.ShapeDtypeStruct((N, H), jnp.float32),
    )(p0, p1, lin2_w, lin2_b.reshape(1, H), lin_w, lin_b.reshape(1, H))
    return out


# speedup vs baseline: 1.6833x; 1.6833x over previous
import functools

import jax
import jax.numpy as jnp
from jax import lax
from jax.experimental import pallas as pl
from jax.experimental.pallas import tpu as pltpu
from jax.experimental.pallas import tpu_sc as plsc

N = 10000
E = 320000
H = 128
NUM_RBF = 16
CUTOFF = 5.0
LOG2 = 0.6931471805599453

NC = 2
NS = 16
L = 16
NW = NC * NS
C = 32
E_PAD = 327680
EPT = E_PAD // NW
NCH = EPT // C
N_PAD = 10240
RPT = N_PAD // NS

BE = 2000
BN = 2000


def _wf_body(ea_ref, ew_ref, w1_ref, b1_ref, w2_ref, b2_ref, o_ref):
    ea = ea_ref[...].astype(jnp.bfloat16)
    v = jnp.dot(ea, w1_ref[...].astype(jnp.bfloat16),
                preferred_element_type=jnp.float32)
    v = v + b1_ref[...]
    v = jnp.maximum(v, 0.0) + jnp.log1p(jnp.exp(-jnp.abs(v))) - LOG2
    wf = jnp.dot(v.astype(jnp.bfloat16), w2_ref[...].astype(jnp.bfloat16),
                 preferred_element_type=jnp.float32)
    wf = wf + b2_ref[...]
    ew = ew_ref[...].reshape(1, BE)
    cut = 0.5 * (jnp.cos(ew * (jnp.pi / CUTOFF)) + 1.0)
    cutb = lax.dot_general(cut, jnp.ones((1, H), jnp.float32),
                           (((0,), (0,)), ((), ())),
                           preferred_element_type=jnp.float32)
    o_ref[...] = wf * cutb


def _h_body(x_ref, w_ref, o_ref):
    o_ref[...] = jnp.dot(x_ref[...], w_ref[...],
                         preferred_element_type=jnp.float32)


def _out_body(p0_ref, p1_ref, w2_ref, b2_ref, wo_ref, bo_ref, o_ref):
    agg = p0_ref[...] + p1_ref[...]
    h2 = jnp.dot(agg, w2_ref[...], preferred_element_type=jnp.float32)
    h2 = jnp.tanh(h2 + b2_ref[...])
    o_ref[...] = jnp.dot(h2, wo_ref[...],
                         preferred_element_type=jnp.float32) + bo_ref[...]


_sc_mesh = plsc.VectorSubcoreMesh(core_axis_name="c", subcore_axis_name="s")


NBUF = 2


@functools.partial(
    pl.kernel,
    out_type=[jax.ShapeDtypeStruct((N_PAD, H), jnp.float32),
              jax.ShapeDtypeStruct((N_PAD, H), jnp.float32)],
    mesh=_sc_mesh,
    scratch_types=[
        pltpu.VMEM((EPT // 128, 128), jnp.int32),
        pltpu.VMEM((EPT // 128, 128), jnp.int32),
        pltpu.VMEM((NBUF, C, H), jnp.float32),
        pltpu.VMEM((NBUF, C, H), jnp.float32),
        pltpu.VMEM((NBUF, C, H), jnp.float32),
        pltpu.VMEM((NBUF, C), jnp.int32),
        pltpu.VMEM_SHARED((N_PAD, H), jnp.float32),
        pltpu.SemaphoreType.DMA((NBUF,)),
        pltpu.SemaphoreType.DMA((NBUF,)),
        pltpu.SemaphoreType.DMA((NBUF,)),
    ],
)
def _sc_scatter(h_hbm, wf_hbm, src_hbm, dst_hbm, out0_hbm, out1_hbm,
                src_v, dst_v, rows_v, wf_v, msg_v, dcur_v, agg_sh,
                gsem, wsem, ssem):
    c = lax.axis_index("c")
    s = lax.axis_index("s")
    w = c * NS + s

    idxr = EPT // 128
    pltpu.sync_copy(src_hbm.at[pl.ds(w * idxr, idxr)], src_v)
    pltpu.sync_copy(dst_hbm.at[pl.ds(w * idxr, idxr)], dst_v)

    cpr = 128 // C

    def idx_slice(v, i):
        return v.at[i // cpr, pl.ds((i % cpr) * C, C)]

    zbuf = rows_v.at[0]

    @pl.loop(0, C)
    def _zero_rows(r):
        for j in range(H // L):
            zbuf.at[pl.ds(r, 1), pl.ds(j * L, L)][...] = jnp.zeros(
                (1, L), jnp.float32)

    base_n = s * RPT
    for k in range(RPT // C):
        pltpu.sync_copy(zbuf, agg_sh.at[pl.ds(base_n + k * C, C)])

    def issue(i, b):
        pltpu.async_copy(h_hbm.at[idx_slice(src_v, i)], rows_v.at[b],
                         gsem.at[b])
        pltpu.async_copy(wf_hbm.at[pl.ds((w * NCH + i) * C, C)],
                         wf_v.at[b], wsem.at[b])

    def wait_in(b):
        pltpu.make_async_copy(h_hbm.at[idx_slice(src_v, 0)], rows_v.at[b],
                              gsem.at[b]).wait()
        pltpu.make_async_copy(wf_hbm.at[pl.ds(0, C)], wf_v.at[b],
                              wsem.at[b]).wait()

    def wait_scatter(b):
        pltpu.make_async_copy(msg_v.at[b], agg_sh.at[dcur_v.at[b]],
                              ssem.at[b]).wait()

    issue(0, 0)
    issue(1, 1)
    plsc.subcore_barrier()

    @pl.loop(0, NCH // NBUF)
    def _edge_step(t):
        i0 = t * NBUF
        for b in range(NBUF):
            i = i0 + b
            wait_in(b)

            @pl.when(t > 0)
            def _():
                wait_scatter(b)

            rb, wb, mb = rows_v.at[b], wf_v.at[b], msg_v.at[b]

            @pl.loop(0, C)
            def _mul_row(r):
                for j in range(H // L):
                    slc = (pl.ds(r, 1), pl.ds(j * L, L))
                    mb.at[*slc][...] = rb.at[*slc][...] * wb.at[*slc][...]

            @pl.when(i + NBUF < NCH)
            def _():
                issue(i + NBUF, b)

            dslc = idx_slice(dst_v, i)
            for j in range(C // L):
                dcur_v.at[b, pl.ds(j * L, L)][...] = \
                    dslc.at[pl.ds(j * L, L)][...]
            pltpu.async_copy(msg_v.at[b], agg_sh.at[dcur_v.at[b]],
                             ssem.at[b], add=True)

    for b in range(NBUF):
        wait_scatter(b)
    plsc.subcore_barrier()

    @pl.when(c == 0)
    def _():
        pltpu.sync_copy(agg_sh.at[pl.ds(base_n, RPT)],
                        out0_hbm.at[pl.ds(base_n, RPT)])

    @pl.when(c == 1)
    def _():
        pltpu.sync_copy(agg_sh.at[pl.ds(base_n, RPT)],
                        out1_hbm.at[pl.ds(base_n, RPT)])


def kernel(x, edge_index, edge_weight, edge_attr, atom_types, seq_neighs,
           lin1_w, fnet_w1, fnet_b1, fnet_w2, fnet_b2, lin2_w, lin2_b,
           lin_w, lin_b):
    pad = E_PAD - E
    src = jnp.pad(edge_index[0], (0, pad)).reshape(E_PAD // 128, 128)
    dst = jnp.pad(edge_index[1], (0, pad),
                  constant_values=N).reshape(E_PAD // 128, 128)
    h = pl.pallas_call(
        _h_body,
        out_shape=jax.ShapeDtypeStruct((N, H), jnp.float32),
    )(x, lin1_w)

    wf = pl.pallas_call(
        _wf_body,
        grid=(E // BE,),
        in_specs=[
            pl.BlockSpec((BE, NUM_RBF), lambda i: (i, 0)),
            pl.BlockSpec((1, 1, BE), lambda i: (i, 0, 0)),
            pl.BlockSpec((NUM_RBF, H), lambda i: (0, 0)),
            pl.BlockSpec((1, H), lambda i: (0, 0)),
            pl.BlockSpec((H, H), lambda i: (0, 0)),
            pl.BlockSpec((1, H), lambda i: (0, 0)),
        ],
        out_specs=pl.BlockSpec((BE, H), lambda i: (i, 0)),
        out_shape=jax.ShapeDtypeStruct((E_PAD, H), jnp.float32),
    )(edge_attr, edge_weight.reshape(E // BE, 1, BE), fnet_w1,
      fnet_b1.reshape(1, H), fnet_w2, fnet_b2.reshape(1, H))

    p0, p1 = _sc_scatter(h, wf, src, dst)

    out = pl.pallas_call(
        _out_body,
        grid=(N // BN,),
        in_specs=[
            pl.BlockSpec((BN, H), lambda i: (i, 0)),
            pl.BlockSpec((BN, H), lambda i: (i, 0)),
            pl.BlockSpec((H, H), lambda i: (0, 0)),
            pl.BlockSpec((1, H), lambda i: (0, 0)),
            pl.BlockSpec((H, H), lambda i: (0, 0)),
            pl.BlockSpec((1, H), lambda i: (0, 0)),
        ],
        out_specs=pl.BlockSpec((BN, H), lambda i: (i, 0)),
        out_shape=jax.ShapeDtypeStruct((N, H), jnp.float32),
    )(p0, p1, lin2_w, lin2_b.reshape(1, H), lin_w, lin_b.reshape(1, H))
    return out
